# baseline (speedup 1.0000x reference)
import jax
import jax.numpy as jnp
from jax import lax
from jax.experimental import pallas as pl
from jax.experimental.pallas import tpu as pltpu
from jax.experimental.pallas import tpu_sc as plsc

NC = 2
NS = 16
NW = NC * NS
L = 16

D = 128
V = 100000
B = 1024
G = 10
Z = 10
ROWS = D // NW

LAMB = 0.01


def _sc_body(rv_hbm, rd_hbm, wids_hbm, dids_hbm, nids_hbm,
             wpT_hbm, docsT_hbm, negsT_hbm, ssq_hbm,
             row_v, idx_v, did_v, out1_v, outz_v, ssq_v):
    wid = lax.axis_index("s") * NC + lax.axis_index("c")

    pltpu.sync_copy(wids_hbm, idx_v)
    for r in range(ROWS):
        d = wid * ROWS + r
        pltpu.sync_copy(rv_hbm.at[d], row_v)

        def gbody(i, c):
            acc = jnp.zeros((L,), jnp.float32)
            for g in range(G):
                idx = idx_v[pl.ds(g * B + i * L, L)]
                acc = acc + plsc.load_gather(row_v, [idx])
            out1_v[pl.ds(i * L, L)] = acc * (1.0 / G)
            return c
        lax.fori_loop(0, B // L, gbody, 0)
        pltpu.sync_copy(out1_v, wpT_hbm.at[d])

    pltpu.sync_copy(nids_hbm, idx_v)
    pltpu.sync_copy(dids_hbm, did_v)
    ssq_tot = jnp.zeros((L,), jnp.float32)
    for r in range(ROWS):
        d = wid * ROWS + r
        pltpu.sync_copy(rd_hbm.at[d], row_v)

        def sbody(i, acc):
            v = row_v[pl.ds(i * L, L)]
            return acc + v * v
        ssq_tot = lax.fori_loop(0, V // L, sbody, ssq_tot)

        def dbody(i, c):
            idx = did_v[pl.ds(i * L, L)]
            out1_v[pl.ds(i * L, L)] = plsc.load_gather(row_v, [idx])
            return c
        lax.fori_loop(0, B // L, dbody, 0)
        pltpu.sync_copy(out1_v, docsT_hbm.at[d])

        def nbody(i, c):
            for z in range(Z):
                idx = idx_v[pl.ds(z * B + i * L, L)]
                outz_v[pl.ds(z * B + i * L, L)] = plsc.load_gather(row_v, [idx])
            return c
        lax.fori_loop(0, B // L, nbody, 0)
        pltpu.sync_copy(outz_v, negsT_hbm.at[d])

    ssq_v[...] = ssq_tot
    pltpu.sync_copy(ssq_v, ssq_hbm.at[wid])


import functools


@functools.cache
def _get_sc_call():
  return pl.kernel(
    _sc_body,
    out_type=(
        jax.ShapeDtypeStruct((D, B), jnp.float32),
        jax.ShapeDtypeStruct((D, B), jnp.float32),
        jax.ShapeDtypeStruct((D, Z * B), jnp.float32),
        jax.ShapeDtypeStruct((NW, L), jnp.float32),
    ),
    mesh=plsc.VectorSubcoreMesh(
        core_axis_name="c", subcore_axis_name="s",
        num_cores=NC, num_subcores=NS),
    compiler_params=pltpu.CompilerParams(needs_layout_passes=False),
    scratch_types=[
        pltpu.VMEM((V,), jnp.float32),
        pltpu.VMEM((G * B,), jnp.int32),
        pltpu.VMEM((B,), jnp.int32),
        pltpu.VMEM((B,), jnp.float32),
        pltpu.VMEM((Z * B,), jnp.float32),
        pltpu.VMEM((L,), jnp.float32),
    ],
  )


def _tc_body(wpT_ref, docsT_ref, negsT_ref, ssq_ref, proj_ref, beta_ref,
             out_ref):
    wpT = wpT_ref[...]
    n2 = jnp.sum(wpT * wpT, axis=0, keepdims=True)
    normedT = wpT / jnp.sqrt(n2)
    tT = jnp.dot(proj_ref[...], normedT,
                 preferred_element_type=jnp.float32)
    mean = jnp.mean(tT, axis=1, keepdims=True)
    var = jnp.sum((tT - mean) ** 2, axis=1, keepdims=True) / (B - 1)
    std = jnp.sqrt(var)
    t = jnp.clip((tT - mean) / jnp.sqrt(std) + beta_ref[...], -1.0, 1.0)

    pos = jnp.sum(t * docsT_ref[...], axis=0, keepdims=True)
    p_pos = jnp.minimum(jax.nn.sigmoid(pos), 0.999)
    acc = Z * jnp.log(p_pos)
    for z in range(Z):
        dz = jnp.sum(t * negsT_ref[:, z * B:(z + 1) * B], axis=0,
                     keepdims=True)
        p = jnp.minimum(jax.nn.sigmoid(dz), 0.999)
        acc = acc + jnp.log(jnp.maximum(1.0 - p, 0.01))

    total = jnp.sum(acc) * ((Z + 1) / (2 * Z))
    reg = jnp.sum(ssq_ref[...]) + jnp.sum(proj_ref[...] * proj_ref[...])
    loss = total / B + LAMB / (2 * B) * reg
    out_ref[...] = jnp.broadcast_to(loss, (1, 1))


def kernel(rv, rd, proj, beta, word_ids, doc_ids, nsample_ids):
    widsT = jnp.transpose(word_ids).astype(jnp.int32).reshape(-1)
    nidsT = jnp.transpose(nsample_ids).astype(jnp.int32).reshape(-1)
    dids = doc_ids.astype(jnp.int32)

    wpT, docsT, negsT, ssq = _get_sc_call()(rv, rd, widsT, dids, nidsT)

    out = pl.pallas_call(
        _tc_body,
        out_shape=jax.ShapeDtypeStruct((1, 1), jnp.float32),
    )(wpT, docsT, negsT, ssq, proj, beta)
    return out[0, 0]


# speedup vs baseline: 1.3709x; 1.3709x over previous
import jax
import jax.numpy as jnp
from jax import lax
from jax.experimental import pallas as pl
from jax.experimental.pallas import tpu as pltpu
from jax.experimental.pallas import tpu_sc as plsc

NC = 2
NS = 16
NW = NC * NS
L = 16

D = 128
V = 100000
B = 1024
G = 10
Z = 10
ROWS = D // NW

LAMB = 0.01


def _sc_body(rv_hbm, rd_hbm, wids_hbm, dids_hbm, nids_hbm,
             wpT_hbm, docsT_hbm, negsT_hbm, ssq_hbm,
             row_v, idx_v, did_v, out1_v, outz_v, ssq_v):
    wid = lax.axis_index("s") * NC + lax.axis_index("c")

    pltpu.sync_copy(wids_hbm, idx_v)
    for r in range(ROWS):
        d = wid * ROWS + r
        pltpu.sync_copy(rv_hbm.at[d], row_v)

        def gbody(i, c):
            for u in range(2):
                base = (2 * i + u) * L
                vs = [plsc.load_gather(row_v, [idx_v[pl.ds(g * B + base, L)]])
                      for g in range(G)]
                while len(vs) > 1:
                    vs = [vs[j] + vs[j + 1] for j in range(0, len(vs) - 1, 2)] \
                        + ([vs[-1]] if len(vs) % 2 else [])
                out1_v[pl.ds(base, L)] = vs[0] * (1.0 / G)
            return c
        lax.fori_loop(0, B // L // 2, gbody, 0)
        pltpu.sync_copy(out1_v, wpT_hbm.at[d])

    pltpu.sync_copy(nids_hbm, idx_v)
    pltpu.sync_copy(dids_hbm, did_v)
    ssq_tot = [jnp.zeros((L,), jnp.float32) for _ in range(5)]
    for r in range(ROWS):
        d = wid * ROWS + r
        pltpu.sync_copy(rd_hbm.at[d], row_v)

        def sbody(i, accs):
            accs = list(accs)
            for u in range(10):
                v = row_v[pl.ds((10 * i + u) * L, L)]
                accs[u % 5] = accs[u % 5] + v * v
            return tuple(accs)
        ssq_tot = list(lax.fori_loop(0, V // L // 10, sbody, tuple(ssq_tot)))

        def dbody(i, c):
            for u in range(4):
                base = (4 * i + u) * L
                idx = did_v[pl.ds(base, L)]
                out1_v[pl.ds(base, L)] = plsc.load_gather(row_v, [idx])
            return c
        lax.fori_loop(0, B // L // 4, dbody, 0)
        pltpu.sync_copy(out1_v, docsT_hbm.at[d])

        def nbody(i, c):
            for z in range(Z):
                for u in range(2):
                    base = z * B + (2 * i + u) * L
                    idx = idx_v[pl.ds(base, L)]
                    outz_v[pl.ds(base, L)] = plsc.load_gather(row_v, [idx])
            return c
        lax.fori_loop(0, B // L // 2, nbody, 0)
        pltpu.sync_copy(outz_v, negsT_hbm.at[d])

    ssq_v[...] = ((ssq_tot[0] + ssq_tot[1]) + (ssq_tot[2] + ssq_tot[3])) \
        + ssq_tot[4]
    pltpu.sync_copy(ssq_v, ssq_hbm.at[wid])


import functools


@functools.cache
def _get_sc_call():
  return pl.kernel(
    _sc_body,
    out_type=(
        jax.ShapeDtypeStruct((D, B), jnp.float32),
        jax.ShapeDtypeStruct((D, B), jnp.float32),
        jax.ShapeDtypeStruct((D, Z * B), jnp.float32),
        jax.ShapeDtypeStruct((NW, L), jnp.float32),
    ),
    mesh=plsc.VectorSubcoreMesh(
        core_axis_name="c", subcore_axis_name="s",
        num_cores=NC, num_subcores=NS),
    compiler_params=pltpu.CompilerParams(needs_layout_passes=False),
    scratch_types=[
        pltpu.VMEM((V,), jnp.float32),
        pltpu.VMEM((G * B,), jnp.int32),
        pltpu.VMEM((B,), jnp.int32),
        pltpu.VMEM((B,), jnp.float32),
        pltpu.VMEM((Z * B,), jnp.float32),
        pltpu.VMEM((L,), jnp.float32),
    ],
  )


def _tc_body(wpT_ref, docsT_ref, negsT_ref, ssq_ref, proj_ref, beta_ref,
             out_ref):
    wpT = wpT_ref[...]
    n2 = jnp.sum(wpT * wpT, axis=0, keepdims=True)
    normedT = wpT / jnp.sqrt(n2)
    tT = jnp.dot(proj_ref[...], normedT,
                 preferred_element_type=jnp.float32)
    mean = jnp.mean(tT, axis=1, keepdims=True)
    var = jnp.sum((tT - mean) ** 2, axis=1, keepdims=True) / (B - 1)
    std = jnp.sqrt(var)
    t = jnp.clip((tT - mean) / jnp.sqrt(std) + beta_ref[...], -1.0, 1.0)

    pos = jnp.sum(t * docsT_ref[...], axis=0, keepdims=True)
    p_pos = jnp.minimum(jax.nn.sigmoid(pos), 0.999)
    acc = Z * jnp.log(p_pos)
    for z in range(Z):
        dz = jnp.sum(t * negsT_ref[:, z * B:(z + 1) * B], axis=0,
                     keepdims=True)
        p = jnp.minimum(jax.nn.sigmoid(dz), 0.999)
        acc = acc + jnp.log(jnp.maximum(1.0 - p, 0.01))

    total = jnp.sum(acc) * ((Z + 1) / (2 * Z))
    reg = jnp.sum(ssq_ref[...]) + jnp.sum(proj_ref[...] * proj_ref[...])
    loss = total / B + LAMB / (2 * B) * reg
    out_ref[...] = jnp.broadcast_to(loss, (1, 1))


def kernel(rv, rd, proj, beta, word_ids, doc_ids, nsample_ids):
    widsT = jnp.transpose(word_ids).astype(jnp.int32).reshape(-1)
    nidsT = jnp.transpose(nsample_ids).astype(jnp.int32).reshape(-1)
    dids = doc_ids.astype(jnp.int32)

    wpT, docsT, negsT, ssq = _get_sc_call()(rv, rd, widsT, dids, nidsT)

    out = pl.pallas_call(
        _tc_body,
        out_shape=jax.ShapeDtypeStruct((1, 1), jnp.float32),
    )(wpT, docsT, negsT, ssq, proj, beta)
    return out[0, 0]
